# Initial kernel scaffold; baseline (speedup 1.0000x reference)
import jax
import jax.numpy as jnp
from jax.experimental import pallas as pl


def kernel(x, edge_index, Wl1, Wr1, b1, Wl2, Wr2, b2, Wl3, Wr3, b3):
    raise NotImplementedError("write your pallas kernel here")


# speedup vs baseline: 6.2286x; 6.2286x over previous
import functools

import jax
import jax.numpy as jnp
from jax import lax
from jax.experimental import pallas as pl
from jax.experimental.pallas import tpu as pltpu
from jax.experimental.pallas import tpu_sc as plsc

_N = 10000
_NP = 10240
_E = 320000
_D = 128
_NC = 2
_HD = _D // _NC
_NS = 16
_EPT = _E // _NS
_CHUNK = 125
_NCHUNK = _EPT // _CHUNK
_RPT = _NP // _NS
_ZR = 128
_CW = 16
_LANES = 16

_f32 = jnp.float32


def _fill_rows(ref, nrows, ncol16, value):
    v = jnp.full((_LANES,), value, _f32)

    def body(k, _):
        i = k // ncol16
        j = k % ncol16
        ref[i, pl.ds(j * _LANES, _LANES)] = v
        return 0

    lax.fori_loop(0, nrows * ncol16, body, 0)


def _segsum_body(with_cnt, *refs):
    if with_cnt:
        (p_hbm, src_hbm, dst_hbm, out_hbm, cnt_hbm,
         src_v, dst_v, rows_v, zbuf_v, ones_v, zc_v, acc_s, cnt_s, sem) = refs
    else:
        (p_hbm, src_hbm, dst_hbm, out_hbm,
         src_v, dst_v, rows_v, zbuf_v, acc_s, sem) = refs

    c = lax.axis_index("c")
    s = lax.axis_index("s")

    _fill_rows(zbuf_v, _ZR, _HD // _LANES, 0.0)
    for k in range(_RPT // _ZR):
        pltpu.sync_copy(zbuf_v, acc_s.at[pl.ds(s * _RPT + k * _ZR, _ZR)])
    if with_cnt:
        @pl.when(c == 0)
        def _():
            _fill_rows(zc_v, _ZR, _CW // _LANES, 0.0)
            for k in range(_RPT // _ZR):
                pltpu.sync_copy(zc_v, cnt_s.at[pl.ds(s * _RPT + k * _ZR, _ZR)])
            _fill_rows(ones_v, _CHUNK, _CW // _LANES, 1.0)

    plsc.subcore_barrier()

    pltpu.sync_copy(src_hbm.at[s], src_v)
    pltpu.sync_copy(dst_hbm.at[s], dst_v)
    my_p = p_hbm.at[c]

    if with_cnt:
        def chunk(j, _):
            pltpu.async_copy(my_p.at[src_v.at[j]], rows_v, sem).wait()
            pltpu.sync_copy(rows_v, acc_s.at[dst_v.at[j]], add=True)

            @pl.when(c == 0)
            def _():
                pltpu.sync_copy(ones_v, cnt_s.at[dst_v.at[j]], add=True)
            return 0
    else:
        def chunk(j, _):
            pltpu.async_copy(my_p.at[src_v.at[j]], rows_v, sem).wait()
            pltpu.sync_copy(rows_v, acc_s.at[dst_v.at[j]], add=True)
            return 0

    lax.fori_loop(0, _NCHUNK, chunk, 0)

    plsc.subcore_barrier()

    pltpu.sync_copy(acc_s.at[pl.ds(s * _RPT, _RPT)],
                    out_hbm.at[c].at[pl.ds(s * _RPT, _RPT)])
    if with_cnt:
        @pl.when((c == 0) & (s == 0))
        def _():
            pltpu.sync_copy(cnt_s, cnt_hbm)


@functools.lru_cache(maxsize=None)
def _make_sc_segsum(with_cnt):
    mesh = plsc.VectorSubcoreMesh(core_axis_name="c", subcore_axis_name="s",
                                  num_cores=_NC, num_subcores=_NS)
    out_type = [jax.ShapeDtypeStruct((_NC, _NP, _HD), _f32)]
    scratch = [
        pltpu.VMEM((_NCHUNK, _CHUNK), jnp.int32),
        pltpu.VMEM((_NCHUNK, _CHUNK), jnp.int32),
        pltpu.VMEM((_CHUNK, _HD), _f32),
        pltpu.VMEM((_ZR, _HD), _f32),
    ]
    if with_cnt:
        out_type.append(jax.ShapeDtypeStruct((_NP, _CW), _f32))
        scratch.append(pltpu.VMEM((_CHUNK, _CW), _f32))
        scratch.append(pltpu.VMEM((_ZR, _CW), _f32))
    scratch.append(pltpu.VMEM_SHARED((_NP, _HD), _f32))
    if with_cnt:
        scratch.append(pltpu.VMEM_SHARED((_NP, _CW), _f32))
    scratch.append(pltpu.SemaphoreType.DMA)

    return pl.kernel(
        functools.partial(_segsum_body, with_cnt),
        out_type=tuple(out_type),
        mesh=mesh,
        scratch_types=tuple(scratch),
        compiler_params=pltpu.CompilerParams(use_tc_tiling_on_sc=False),
        name="sc_segsum_cnt" if with_cnt else "sc_segsum",
    )


_BR = 512


def _split(p):
    return jnp.stack([p[:, :_HD], p[:, _HD:]])


def _tc_in_body(x_ref, wl_ref, wr_ref, b_ref, p_ref, q_ref):
    x = x_ref[...]
    p_ref[...] = _split(jnp.dot(x, wl_ref[...], preferred_element_type=_f32))
    q_ref[...] = jnp.dot(x, wr_ref[...], preferred_element_type=_f32) + b_ref[...]


def _tc_in(x, wl, wr, b):
    return pl.pallas_call(
        _tc_in_body,
        grid=(_NP // _BR,),
        in_specs=[
            pl.BlockSpec((_BR, _D), lambda i: (i, 0)),
            pl.BlockSpec((_D, _D), lambda i: (0, 0)),
            pl.BlockSpec((_D, _D), lambda i: (0, 0)),
            pl.BlockSpec((1, _D), lambda i: (0, 0)),
        ],
        out_specs=[
            pl.BlockSpec((_NC, _BR, _HD), lambda i: (0, i, 0)),
            pl.BlockSpec((_BR, _D), lambda i: (i, 0)),
        ],
        out_shape=[
            jax.ShapeDtypeStruct((_NC, _NP, _HD), _f32),
            jax.ShapeDtypeStruct((_NP, _D), _f32),
        ],
    )(x, wl, wr, b.reshape(1, _D))


def _relu_mean(s_ref, cnt_ref, q_ref):
    ssum = jnp.concatenate([s_ref[0], s_ref[1]], axis=1)
    cnt = cnt_ref[:, 0:1]
    mean = ssum / jnp.maximum(cnt, 1.0)
    return jnp.maximum(mean + q_ref[...], 0.0)


def _tc_mid_body(s_ref, cnt_ref, q_ref, wl_ref, wr_ref, b_ref, p_ref, q2_ref):
    h = _relu_mean(s_ref, cnt_ref, q_ref)
    p_ref[...] = _split(jnp.dot(h, wl_ref[...], preferred_element_type=_f32))
    q2_ref[...] = jnp.dot(h, wr_ref[...], preferred_element_type=_f32) + b_ref[...]


def _tc_mid(s, cnt, q, wl, wr, b):
    return pl.pallas_call(
        _tc_mid_body,
        grid=(_NP // _BR,),
        in_specs=[
            pl.BlockSpec((_NC, _BR, _HD), lambda i: (0, i, 0)),
            pl.BlockSpec((_BR, _CW), lambda i: (i, 0)),
            pl.BlockSpec((_BR, _D), lambda i: (i, 0)),
            pl.BlockSpec((_D, _D), lambda i: (0, 0)),
            pl.BlockSpec((_D, _D), lambda i: (0, 0)),
            pl.BlockSpec((1, _D), lambda i: (0, 0)),
        ],
        out_specs=[
            pl.BlockSpec((_NC, _BR, _HD), lambda i: (0, i, 0)),
            pl.BlockSpec((_BR, _D), lambda i: (i, 0)),
        ],
        out_shape=[
            jax.ShapeDtypeStruct((_NC, _NP, _HD), _f32),
            jax.ShapeDtypeStruct((_NP, _D), _f32),
        ],
    )(s, cnt, q, wl, wr, b.reshape(1, _D))


def _tc_out_body(s_ref, cnt_ref, q_ref, h_ref):
    h_ref[...] = _relu_mean(s_ref, cnt_ref, q_ref)


def _tc_out(s, cnt, q):
    return pl.pallas_call(
        _tc_out_body,
        grid=(_NP // _BR,),
        in_specs=[
            pl.BlockSpec((_NC, _BR, _HD), lambda i: (0, i, 0)),
            pl.BlockSpec((_BR, _CW), lambda i: (i, 0)),
            pl.BlockSpec((_BR, _D), lambda i: (i, 0)),
        ],
        out_specs=pl.BlockSpec((_BR, _D), lambda i: (i, 0)),
        out_shape=jax.ShapeDtypeStruct((_NP, _D), _f32),
    )(s, cnt, q)


def kernel(x, edge_index, Wl1, Wr1, b1, Wl2, Wr2, b2, Wl3, Wr3, b3):
    src = edge_index[0].astype(jnp.int32).reshape(_NS, _NCHUNK, _CHUNK)
    dst = edge_index[1].astype(jnp.int32).reshape(_NS, _NCHUNK, _CHUNK)
    xp = jnp.pad(x, ((0, _NP - _N), (0, 0)))

    p1, q1 = _tc_in(xp, Wl1, Wr1, b1)
    s1, cnt = _make_sc_segsum(True)(p1, src, dst)
    p2, q2 = _tc_mid(s1, cnt, q1, Wl2, Wr2, b2)
    (s2,) = _make_sc_segsum(False)(p2, src, dst)
    p3, q3 = _tc_mid(s2, cnt, q2, Wl3, Wr3, b3)
    (s3,) = _make_sc_segsum(False)(p3, src, dst)
    return _tc_out(s3, cnt, q3)[:_N]


# speedup vs baseline: 9.6427x; 1.5481x over previous
import functools

import jax
import jax.numpy as jnp
from jax import lax
from jax.experimental import pallas as pl
from jax.experimental.pallas import tpu as pltpu
from jax.experimental.pallas import tpu_sc as plsc

_N = 10000
_NP = 10240
_E = 320000
_D = 128
_NC = 2
_HD = _D // _NC
_NS = 16
_EPT = _E // _NS
_CHUNK = 125
_NCHUNK = _EPT // _CHUNK
_RPT = _NP // _NS
_ZR = 128
_CW = 16
_LANES = 16

_f32 = jnp.float32


def _fill_rows(ref, nrows, ncol16, value):
    v = jnp.full((_LANES,), value, _f32)

    def body(k, _):
        i = k // ncol16
        j = k % ncol16
        ref[i, pl.ds(j * _LANES, _LANES)] = v
        return 0

    lax.fori_loop(0, nrows * ncol16, body, 0)


def _segsum_body(with_cnt, *refs):
    if with_cnt:
        (p_hbm, src_hbm, dst_hbm, out_hbm, cnt_hbm,
         src_v, dst_v, rows_a, rows_b, zbuf_v, ones_v, zc_v,
         acc_s, cnt_s, sem_a, sem_b) = refs
    else:
        (p_hbm, src_hbm, dst_hbm, out_hbm,
         src_v, dst_v, rows_a, rows_b, zbuf_v, acc_s, sem_a, sem_b) = refs

    c = lax.axis_index("c")
    s = lax.axis_index("s")

    _fill_rows(zbuf_v, _ZR, _HD // _LANES, 0.0)
    for k in range(_RPT // _ZR):
        pltpu.sync_copy(zbuf_v, acc_s.at[pl.ds(s * _RPT + k * _ZR, _ZR)])
    if with_cnt:
        @pl.when(c == 0)
        def _():
            _fill_rows(zc_v, _ZR, _CW // _LANES, 0.0)
            for k in range(_RPT // _ZR):
                pltpu.sync_copy(zc_v, cnt_s.at[pl.ds(s * _RPT + k * _ZR, _ZR)])
            _fill_rows(ones_v, _CHUNK, _CW // _LANES, 1.0)

    plsc.subcore_barrier()

    pltpu.sync_copy(src_hbm.at[s], src_v)
    pltpu.sync_copy(dst_hbm.at[s], dst_v)
    my_p = p_hbm.at[c]

    def gath(j, buf, sem):
        pltpu.async_copy(my_p.at[src_v.at[j]], buf, sem)

    def gwait(buf, sem):
        pltpu.make_async_copy(my_p.at[src_v.at[0]], buf, sem).wait()

    def scat(j, buf):
        pltpu.sync_copy(buf, acc_s.at[dst_v.at[j]], add=True)
        if with_cnt:
            @pl.when(c == 0)
            def _():
                pltpu.sync_copy(ones_v, cnt_s.at[dst_v.at[j]], add=True)

    gath(0, rows_a, sem_a)

    def body(k, _):
        j = 2 * k
        gath(j + 1, rows_b, sem_b)
        gwait(rows_a, sem_a)
        scat(j, rows_a)
        gath(j + 2, rows_a, sem_a)
        gwait(rows_b, sem_b)
        scat(j + 1, rows_b)
        return 0

    lax.fori_loop(0, _NCHUNK // 2 - 1, body, 0)
    j = _NCHUNK - 2
    gath(j + 1, rows_b, sem_b)
    gwait(rows_a, sem_a)
    scat(j, rows_a)
    gwait(rows_b, sem_b)
    scat(j + 1, rows_b)

    plsc.subcore_barrier()

    pltpu.sync_copy(acc_s.at[pl.ds(s * _RPT, _RPT)],
                    out_hbm.at[c].at[pl.ds(s * _RPT, _RPT)])
    if with_cnt:
        @pl.when((c == 0) & (s == 0))
        def _():
            pltpu.sync_copy(cnt_s, cnt_hbm)


@functools.lru_cache(maxsize=None)
def _make_sc_segsum(with_cnt):
    mesh = plsc.VectorSubcoreMesh(core_axis_name="c", subcore_axis_name="s",
                                  num_cores=_NC, num_subcores=_NS)
    out_type = [jax.ShapeDtypeStruct((_NC, _NP, _HD), _f32)]
    scratch = [
        pltpu.VMEM((_NCHUNK, _CHUNK), jnp.int32),
        pltpu.VMEM((_NCHUNK, _CHUNK), jnp.int32),
        pltpu.VMEM((_CHUNK, _HD), _f32),
        pltpu.VMEM((_CHUNK, _HD), _f32),
        pltpu.VMEM((_ZR, _HD), _f32),
    ]
    if with_cnt:
        out_type.append(jax.ShapeDtypeStruct((_NP, _CW), _f32))
        scratch.append(pltpu.VMEM((_CHUNK, _CW), _f32))
        scratch.append(pltpu.VMEM((_ZR, _CW), _f32))
    scratch.append(pltpu.VMEM_SHARED((_NP, _HD), _f32))
    if with_cnt:
        scratch.append(pltpu.VMEM_SHARED((_NP, _CW), _f32))
    scratch.append(pltpu.SemaphoreType.DMA)
    scratch.append(pltpu.SemaphoreType.DMA)

    return pl.kernel(
        functools.partial(_segsum_body, with_cnt),
        out_type=tuple(out_type),
        mesh=mesh,
        scratch_types=tuple(scratch),
        compiler_params=pltpu.CompilerParams(use_tc_tiling_on_sc=False),
        name="sc_segsum_cnt" if with_cnt else "sc_segsum",
    )


_BR = 512


def _split(p):
    return jnp.stack([p[:, :_HD], p[:, _HD:]])


def _tc_in_body(x_ref, wl_ref, wr_ref, b_ref, p_ref, q_ref):
    x = x_ref[...]
    p_ref[...] = _split(jnp.dot(x, wl_ref[...], preferred_element_type=_f32))
    q_ref[...] = jnp.dot(x, wr_ref[...], preferred_element_type=_f32) + b_ref[...]


def _tc_in(x, wl, wr, b):
    return pl.pallas_call(
        _tc_in_body,
        grid=(_NP // _BR,),
        in_specs=[
            pl.BlockSpec((_BR, _D), lambda i: (i, 0)),
            pl.BlockSpec((_D, _D), lambda i: (0, 0)),
            pl.BlockSpec((_D, _D), lambda i: (0, 0)),
            pl.BlockSpec((1, _D), lambda i: (0, 0)),
        ],
        out_specs=[
            pl.BlockSpec((_NC, _BR, _HD), lambda i: (0, i, 0)),
            pl.BlockSpec((_BR, _D), lambda i: (i, 0)),
        ],
        out_shape=[
            jax.ShapeDtypeStruct((_NC, _NP, _HD), _f32),
            jax.ShapeDtypeStruct((_NP, _D), _f32),
        ],
    )(x, wl, wr, b.reshape(1, _D))


def _relu_mean(s_ref, cnt_ref, q_ref):
    ssum = jnp.concatenate([s_ref[0], s_ref[1]], axis=1)
    cnt = cnt_ref[:, 0:1]
    mean = ssum / jnp.maximum(cnt, 1.0)
    return jnp.maximum(mean + q_ref[...], 0.0)


def _tc_mid_body(s_ref, cnt_ref, q_ref, wl_ref, wr_ref, b_ref, p_ref, q2_ref):
    h = _relu_mean(s_ref, cnt_ref, q_ref)
    p_ref[...] = _split(jnp.dot(h, wl_ref[...], preferred_element_type=_f32))
    q2_ref[...] = jnp.dot(h, wr_ref[...], preferred_element_type=_f32) + b_ref[...]


def _tc_mid(s, cnt, q, wl, wr, b):
    return pl.pallas_call(
        _tc_mid_body,
        grid=(_NP // _BR,),
        in_specs=[
            pl.BlockSpec((_NC, _BR, _HD), lambda i: (0, i, 0)),
            pl.BlockSpec((_BR, _CW), lambda i: (i, 0)),
            pl.BlockSpec((_BR, _D), lambda i: (i, 0)),
            pl.BlockSpec((_D, _D), lambda i: (0, 0)),
            pl.BlockSpec((_D, _D), lambda i: (0, 0)),
            pl.BlockSpec((1, _D), lambda i: (0, 0)),
        ],
        out_specs=[
            pl.BlockSpec((_NC, _BR, _HD), lambda i: (0, i, 0)),
            pl.BlockSpec((_BR, _D), lambda i: (i, 0)),
        ],
        out_shape=[
            jax.ShapeDtypeStruct((_NC, _NP, _HD), _f32),
            jax.ShapeDtypeStruct((_NP, _D), _f32),
        ],
    )(s, cnt, q, wl, wr, b.reshape(1, _D))


def _tc_out_body(s_ref, cnt_ref, q_ref, h_ref):
    h_ref[...] = _relu_mean(s_ref, cnt_ref, q_ref)


def _tc_out(s, cnt, q):
    return pl.pallas_call(
        _tc_out_body,
        grid=(_NP // _BR,),
        in_specs=[
            pl.BlockSpec((_NC, _BR, _HD), lambda i: (0, i, 0)),
            pl.BlockSpec((_BR, _CW), lambda i: (i, 0)),
            pl.BlockSpec((_BR, _D), lambda i: (i, 0)),
        ],
        out_specs=pl.BlockSpec((_BR, _D), lambda i: (i, 0)),
        out_shape=jax.ShapeDtypeStruct((_NP, _D), _f32),
    )(s, cnt, q)


def kernel(x, edge_index, Wl1, Wr1, b1, Wl2, Wr2, b2, Wl3, Wr3, b3):
    src = edge_index[0].astype(jnp.int32).reshape(_NS, _NCHUNK, _CHUNK)
    dst = edge_index[1].astype(jnp.int32).reshape(_NS, _NCHUNK, _CHUNK)
    xp = jnp.pad(x, ((0, _NP - _N), (0, 0)))

    p1, q1 = _tc_in(xp, Wl1, Wr1, b1)
    s1, cnt = _make_sc_segsum(True)(p1, src, dst)
    p2, q2 = _tc_mid(s1, cnt, q1, Wl2, Wr2, b2)
    (s2,) = _make_sc_segsum(False)(p2, src, dst)
    p3, q3 = _tc_mid(s2, cnt, q2, Wl3, Wr3, b3)
    (s3,) = _make_sc_segsum(False)(p3, src, dst)
    return _tc_out(s3, cnt, q3)[:_N]


# speedup vs baseline: 10.8927x; 1.1296x over previous
import functools

import jax
import jax.numpy as jnp
from jax import lax
from jax.experimental import pallas as pl
from jax.experimental.pallas import tpu as pltpu
from jax.experimental.pallas import tpu_sc as plsc

_N = 10000
_NP = 10240
_E = 320000
_D = 128
_NC = 2
_HD = _D // _NC
_NS = 16
_EPT = _E // _NS
_CHUNK = 125
_NCHUNK = _EPT // _CHUNK
_RPT = _NP // _NS
_ZR = 128
_CW = 16
_LANES = 16

_f32 = jnp.float32


def _nbuf(with_cnt):
    return 2 if with_cnt else 4


def _fill_rows(ref, nrows, ncol16, value):
    v = jnp.full((_LANES,), value, _f32)

    def body(k, _):
        i = k // ncol16
        j = k % ncol16
        ref[i, pl.ds(j * _LANES, _LANES)] = v
        return 0

    lax.fori_loop(0, nrows * ncol16, body, 0)


def _segsum_body(with_cnt, *refs):
    nbuf = _nbuf(with_cnt)
    if with_cnt:
        (p_hbm, src_hbm, dst_hbm, out_hbm, cnt_hbm,
         src_v, dst_v, *rest) = refs
        rows = rest[:nbuf]
        zbuf_v, ones_v, zc_v, acc_s, cnt_s = rest[nbuf:nbuf + 5]
        sems = rest[nbuf + 5:]
    else:
        (p_hbm, src_hbm, dst_hbm, out_hbm,
         src_v, dst_v, *rest) = refs
        rows = rest[:nbuf]
        zbuf_v, acc_s = rest[nbuf:nbuf + 2]
        sems = rest[nbuf + 2:]
    bufs = tuple(zip(rows, sems))

    c = lax.axis_index("c")
    s = lax.axis_index("s")

    _fill_rows(zbuf_v, _ZR, _HD // _LANES, 0.0)
    for k in range(_RPT // _ZR):
        pltpu.sync_copy(zbuf_v, acc_s.at[pl.ds(s * _RPT + k * _ZR, _ZR)])
    if with_cnt:
        @pl.when(c == 0)
        def _():
            _fill_rows(zc_v, _ZR, _CW // _LANES, 0.0)
            for k in range(_RPT // _ZR):
                pltpu.sync_copy(zc_v, cnt_s.at[pl.ds(s * _RPT + k * _ZR, _ZR)])
            _fill_rows(ones_v, _CHUNK, _CW // _LANES, 1.0)

    plsc.subcore_barrier()

    pltpu.sync_copy(src_hbm.at[s], src_v)
    pltpu.sync_copy(dst_hbm.at[s], dst_v)
    my_p = p_hbm.at[c]

    def gath(j, b):
        pltpu.async_copy(my_p.at[src_v.at[j]], bufs[b][0], bufs[b][1])

    def gwait(b):
        pltpu.make_async_copy(my_p.at[src_v.at[0]], bufs[b][0], bufs[b][1]).wait()

    def scat(j, b):
        pltpu.sync_copy(bufs[b][0], acc_s.at[dst_v.at[j]], add=True)
        if with_cnt:
            @pl.when(c == 0)
            def _():
                pltpu.sync_copy(ones_v, cnt_s.at[dst_v.at[j]], add=True)

    for b in range(nbuf - 1):
        gath(b, b)

    def body(k, _):
        j = k * nbuf
        for u in range(nbuf):
            gath(j + nbuf - 1 + u, (nbuf - 1 + u) % nbuf)
            gwait(u)
            scat(j + u, u)
        return 0

    lax.fori_loop(0, (_NCHUNK - nbuf) // nbuf, body, 0)
    j = _NCHUNK - nbuf
    gath(j + nbuf - 1, nbuf - 1)
    for u in range(nbuf):
        gwait(u)
        scat(j + u, u)

    plsc.subcore_barrier()

    pltpu.sync_copy(acc_s.at[pl.ds(s * _RPT, _RPT)],
                    out_hbm.at[c].at[pl.ds(s * _RPT, _RPT)])
    if with_cnt:
        @pl.when((c == 0) & (s == 0))
        def _():
            pltpu.sync_copy(cnt_s, cnt_hbm)


@functools.lru_cache(maxsize=None)
def _make_sc_segsum(with_cnt):
    mesh = plsc.VectorSubcoreMesh(core_axis_name="c", subcore_axis_name="s",
                                  num_cores=_NC, num_subcores=_NS)
    nbuf = _nbuf(with_cnt)
    out_type = [jax.ShapeDtypeStruct((_NC, _NP, _HD), _f32)]
    scratch = [
        pltpu.VMEM((_NCHUNK, _CHUNK), jnp.int32),
        pltpu.VMEM((_NCHUNK, _CHUNK), jnp.int32),
    ]
    for _ in range(nbuf):
        scratch.append(pltpu.VMEM((_CHUNK, _HD), _f32))
    scratch.append(pltpu.VMEM((_ZR, _HD), _f32))
    if with_cnt:
        out_type.append(jax.ShapeDtypeStruct((_NP, _CW), _f32))
        scratch.append(pltpu.VMEM((_CHUNK, _CW), _f32))
        scratch.append(pltpu.VMEM((_ZR, _CW), _f32))
    scratch.append(pltpu.VMEM_SHARED((_NP, _HD), _f32))
    if with_cnt:
        scratch.append(pltpu.VMEM_SHARED((_NP, _CW), _f32))
    for _ in range(nbuf):
        scratch.append(pltpu.SemaphoreType.DMA)

    return pl.kernel(
        functools.partial(_segsum_body, with_cnt),
        out_type=tuple(out_type),
        mesh=mesh,
        scratch_types=tuple(scratch),
        compiler_params=pltpu.CompilerParams(use_tc_tiling_on_sc=False),
        name="sc_segsum_cnt" if with_cnt else "sc_segsum",
    )


_BR = 512


def _split(p):
    return jnp.stack([p[:, :_HD], p[:, _HD:]])


def _tc_in_body(x_ref, wl_ref, wr_ref, b_ref, p_ref, q_ref):
    x = x_ref[...]
    p_ref[...] = _split(jnp.dot(x, wl_ref[...], preferred_element_type=_f32))
    q_ref[...] = jnp.dot(x, wr_ref[...], preferred_element_type=_f32) + b_ref[...]


def _tc_in(x, wl, wr, b):
    return pl.pallas_call(
        _tc_in_body,
        grid=(_NP // _BR,),
        in_specs=[
            pl.BlockSpec((_BR, _D), lambda i: (i, 0)),
            pl.BlockSpec((_D, _D), lambda i: (0, 0)),
            pl.BlockSpec((_D, _D), lambda i: (0, 0)),
            pl.BlockSpec((1, _D), lambda i: (0, 0)),
        ],
        out_specs=[
            pl.BlockSpec((_NC, _BR, _HD), lambda i: (0, i, 0)),
            pl.BlockSpec((_BR, _D), lambda i: (i, 0)),
        ],
        out_shape=[
            jax.ShapeDtypeStruct((_NC, _NP, _HD), _f32),
            jax.ShapeDtypeStruct((_NP, _D), _f32),
        ],
    )(x, wl, wr, b.reshape(1, _D))


def _relu_mean(s_ref, cnt_ref, q_ref):
    ssum = jnp.concatenate([s_ref[0], s_ref[1]], axis=1)
    cnt = cnt_ref[:, 0:1]
    mean = ssum / jnp.maximum(cnt, 1.0)
    return jnp.maximum(mean + q_ref[...], 0.0)


def _tc_mid_body(s_ref, cnt_ref, q_ref, wl_ref, wr_ref, b_ref, p_ref, q2_ref):
    h = _relu_mean(s_ref, cnt_ref, q_ref)
    p_ref[...] = _split(jnp.dot(h, wl_ref[...], preferred_element_type=_f32))
    q2_ref[...] = jnp.dot(h, wr_ref[...], preferred_element_type=_f32) + b_ref[...]


def _tc_mid(s, cnt, q, wl, wr, b):
    return pl.pallas_call(
        _tc_mid_body,
        grid=(_NP // _BR,),
        in_specs=[
            pl.BlockSpec((_NC, _BR, _HD), lambda i: (0, i, 0)),
            pl.BlockSpec((_BR, _CW), lambda i: (i, 0)),
            pl.BlockSpec((_BR, _D), lambda i: (i, 0)),
            pl.BlockSpec((_D, _D), lambda i: (0, 0)),
            pl.BlockSpec((_D, _D), lambda i: (0, 0)),
            pl.BlockSpec((1, _D), lambda i: (0, 0)),
        ],
        out_specs=[
            pl.BlockSpec((_NC, _BR, _HD), lambda i: (0, i, 0)),
            pl.BlockSpec((_BR, _D), lambda i: (i, 0)),
        ],
        out_shape=[
            jax.ShapeDtypeStruct((_NC, _NP, _HD), _f32),
            jax.ShapeDtypeStruct((_NP, _D), _f32),
        ],
    )(s, cnt, q, wl, wr, b.reshape(1, _D))


def _tc_out_body(s_ref, cnt_ref, q_ref, h_ref):
    h_ref[...] = _relu_mean(s_ref, cnt_ref, q_ref)


def _tc_out(s, cnt, q):
    return pl.pallas_call(
        _tc_out_body,
        grid=(_NP // _BR,),
        in_specs=[
            pl.BlockSpec((_NC, _BR, _HD), lambda i: (0, i, 0)),
            pl.BlockSpec((_BR, _CW), lambda i: (i, 0)),
            pl.BlockSpec((_BR, _D), lambda i: (i, 0)),
        ],
        out_specs=pl.BlockSpec((_BR, _D), lambda i: (i, 0)),
        out_shape=jax.ShapeDtypeStruct((_NP, _D), _f32),
    )(s, cnt, q)


def kernel(x, edge_index, Wl1, Wr1, b1, Wl2, Wr2, b2, Wl3, Wr3, b3):
    src = edge_index[0].astype(jnp.int32).reshape(_NS, _NCHUNK, _CHUNK)
    dst = edge_index[1].astype(jnp.int32).reshape(_NS, _NCHUNK, _CHUNK)
    xp = jnp.pad(x, ((0, _NP - _N), (0, 0)))

    p1, q1 = _tc_in(xp, Wl1, Wr1, b1)
    s1, cnt = _make_sc_segsum(True)(p1, src, dst)
    p2, q2 = _tc_mid(s1, cnt, q1, Wl2, Wr2, b2)
    (s2,) = _make_sc_segsum(False)(p2, src, dst)
    p3, q3 = _tc_mid(s2, cnt, q2, Wl3, Wr3, b3)
    (s3,) = _make_sc_segsum(False)(p3, src, dst)
    return _tc_out(s3, cnt, q3)[:_N]


# speedup vs baseline: 11.0075x; 1.0105x over previous
import functools

import jax
import jax.numpy as jnp
from jax import lax
from jax.experimental import pallas as pl
from jax.experimental.pallas import tpu as pltpu
from jax.experimental.pallas import tpu_sc as plsc

_N = 10000
_NP = 10240
_E = 320000
_D = 128
_NC = 2
_HD = _D // _NC
_NS = 16
_EPT = _E // _NS
_CHUNK = 125
_NCHUNK = _EPT // _CHUNK
_RPT = _NP // _NS
_ZR = 128
_CW = 16
_LANES = 16

_f32 = jnp.float32


def _nbuf(with_cnt):
    return 2 if with_cnt else 4


def _fill_rows(ref, nrows, ncol16, value):
    v = jnp.full((_LANES,), value, _f32)

    def body(k, _):
        i = k // ncol16
        j = k % ncol16
        ref[i, pl.ds(j * _LANES, _LANES)] = v
        return 0

    lax.fori_loop(0, nrows * ncol16, body, 0)


def _segsum_body(with_cnt, *refs):
    nbuf = _nbuf(with_cnt)
    if with_cnt:
        (p_hbm, src_hbm, dst_hbm, zo_hbm, out_hbm, cnt_hbm,
         src_v, dst_v, *rest) = refs
        rows = rest[:nbuf]
        zbuf_v, zo_v, acc_s, cnt_s = rest[nbuf:nbuf + 4]
        sems = rest[nbuf + 4:]
    else:
        (p_hbm, src_hbm, dst_hbm, out_hbm,
         src_v, dst_v, *rest) = refs
        rows = rest[:nbuf]
        zbuf_v, acc_s = rest[nbuf:nbuf + 2]
        sems = rest[nbuf + 2:]
    bufs = tuple(zip(rows, sems))

    c = lax.axis_index("c")
    s = lax.axis_index("s")

    _fill_rows(zbuf_v, _ZR, _HD // _LANES, 0.0)
    for k in range(_RPT // _ZR):
        pltpu.sync_copy(zbuf_v, acc_s.at[pl.ds(s * _RPT + k * _ZR, _ZR)])
    if with_cnt:
        pltpu.sync_copy(zo_hbm, zo_v)
        ones_v = zo_v.at[pl.ds(_ZR, _CHUNK)]

        @pl.when(c == 0)
        def _():
            for k in range(_RPT // _ZR):
                pltpu.sync_copy(zo_v.at[pl.ds(0, _ZR)],
                                cnt_s.at[pl.ds(s * _RPT + k * _ZR, _ZR)])

    plsc.subcore_barrier()

    pltpu.sync_copy(src_hbm.at[s], src_v)
    pltpu.sync_copy(dst_hbm.at[s], dst_v)
    my_p = p_hbm.at[c]

    def gath(j, b):
        pltpu.async_copy(my_p.at[src_v.at[j]], bufs[b][0], bufs[b][1])

    def gwait(b):
        pltpu.make_async_copy(my_p.at[src_v.at[0]], bufs[b][0], bufs[b][1]).wait()

    def scat(j, b):
        pltpu.sync_copy(bufs[b][0], acc_s.at[dst_v.at[j]], add=True)
        if with_cnt:
            @pl.when(c == 0)
            def _():
                pltpu.sync_copy(ones_v, cnt_s.at[dst_v.at[j]], add=True)

    for b in range(nbuf - 1):
        gath(b, b)

    def body(k, _):
        j = k * nbuf
        for u in range(nbuf):
            gath(j + nbuf - 1 + u, (nbuf - 1 + u) % nbuf)
            gwait(u)
            scat(j + u, u)
        return 0

    lax.fori_loop(0, (_NCHUNK - nbuf) // nbuf, body, 0)
    j = _NCHUNK - nbuf
    gath(j + nbuf - 1, nbuf - 1)
    for u in range(nbuf):
        gwait(u)
        scat(j + u, u)

    plsc.subcore_barrier()

    pltpu.sync_copy(acc_s.at[pl.ds(s * _RPT, _RPT)],
                    out_hbm.at[c].at[pl.ds(s * _RPT, _RPT)])
    if with_cnt:
        @pl.when((c == 0) & (s == 0))
        def _():
            pltpu.sync_copy(cnt_s, cnt_hbm)


@functools.lru_cache(maxsize=None)
def _make_sc_segsum(with_cnt):
    mesh = plsc.VectorSubcoreMesh(core_axis_name="c", subcore_axis_name="s",
                                  num_cores=_NC, num_subcores=_NS)
    nbuf = _nbuf(with_cnt)
    out_type = [jax.ShapeDtypeStruct((_NC, _NP, _HD), _f32)]
    scratch = [
        pltpu.VMEM((_NCHUNK, _CHUNK), jnp.int32),
        pltpu.VMEM((_NCHUNK, _CHUNK), jnp.int32),
    ]
    for _ in range(nbuf):
        scratch.append(pltpu.VMEM((_CHUNK, _HD), _f32))
    scratch.append(pltpu.VMEM((_ZR, _HD), _f32))
    if with_cnt:
        out_type.append(jax.ShapeDtypeStruct((_NP, _CW), _f32))
        scratch.append(pltpu.VMEM((_ZR + _CHUNK, _CW), _f32))
    scratch.append(pltpu.VMEM_SHARED((_NP, _HD), _f32))
    if with_cnt:
        scratch.append(pltpu.VMEM_SHARED((_NP, _CW), _f32))
    for _ in range(nbuf):
        scratch.append(pltpu.SemaphoreType.DMA)

    return pl.kernel(
        functools.partial(_segsum_body, with_cnt),
        out_type=tuple(out_type),
        mesh=mesh,
        scratch_types=tuple(scratch),
        compiler_params=pltpu.CompilerParams(use_tc_tiling_on_sc=False),
        name="sc_segsum_cnt" if with_cnt else "sc_segsum",
    )


_BR = 512


def _split(p):
    return jnp.stack([p[:, :_HD], p[:, _HD:]])


def _tc_in_body(x_ref, wl_ref, wr_ref, b_ref, p_ref, q_ref):
    x = x_ref[...]
    p_ref[...] = _split(jnp.dot(x, wl_ref[...], preferred_element_type=_f32))
    q_ref[...] = jnp.dot(x, wr_ref[...], preferred_element_type=_f32) + b_ref[...]


def _tc_in(x, wl, wr, b):
    return pl.pallas_call(
        _tc_in_body,
        grid=(_NP // _BR,),
        in_specs=[
            pl.BlockSpec((_BR, _D), lambda i: (i, 0)),
            pl.BlockSpec((_D, _D), lambda i: (0, 0)),
            pl.BlockSpec((_D, _D), lambda i: (0, 0)),
            pl.BlockSpec((1, _D), lambda i: (0, 0)),
        ],
        out_specs=[
            pl.BlockSpec((_NC, _BR, _HD), lambda i: (0, i, 0)),
            pl.BlockSpec((_BR, _D), lambda i: (i, 0)),
        ],
        out_shape=[
            jax.ShapeDtypeStruct((_NC, _NP, _HD), _f32),
            jax.ShapeDtypeStruct((_NP, _D), _f32),
        ],
    )(x, wl, wr, b.reshape(1, _D))


def _relu_mean(s_ref, cnt_ref, q_ref):
    ssum = jnp.concatenate([s_ref[0], s_ref[1]], axis=1)
    cnt = cnt_ref[:, 0:1]
    mean = ssum / jnp.maximum(cnt, 1.0)
    return jnp.maximum(mean + q_ref[...], 0.0)


def _tc_mid_body(s_ref, cnt_ref, q_ref, wl_ref, wr_ref, b_ref, p_ref, q2_ref):
    h = _relu_mean(s_ref, cnt_ref, q_ref)
    p_ref[...] = _split(jnp.dot(h, wl_ref[...], preferred_element_type=_f32))
    q2_ref[...] = jnp.dot(h, wr_ref[...], preferred_element_type=_f32) + b_ref[...]


def _tc_mid(s, cnt, q, wl, wr, b):
    return pl.pallas_call(
        _tc_mid_body,
        grid=(_NP // _BR,),
        in_specs=[
            pl.BlockSpec((_NC, _BR, _HD), lambda i: (0, i, 0)),
            pl.BlockSpec((_BR, _CW), lambda i: (i, 0)),
            pl.BlockSpec((_BR, _D), lambda i: (i, 0)),
            pl.BlockSpec((_D, _D), lambda i: (0, 0)),
            pl.BlockSpec((_D, _D), lambda i: (0, 0)),
            pl.BlockSpec((1, _D), lambda i: (0, 0)),
        ],
        out_specs=[
            pl.BlockSpec((_NC, _BR, _HD), lambda i: (0, i, 0)),
            pl.BlockSpec((_BR, _D), lambda i: (i, 0)),
        ],
        out_shape=[
            jax.ShapeDtypeStruct((_NC, _NP, _HD), _f32),
            jax.ShapeDtypeStruct((_NP, _D), _f32),
        ],
    )(s, cnt, q, wl, wr, b.reshape(1, _D))


def _tc_out_body(s_ref, cnt_ref, q_ref, h_ref):
    h_ref[...] = _relu_mean(s_ref, cnt_ref, q_ref)


def _tc_out(s, cnt, q):
    return pl.pallas_call(
        _tc_out_body,
        grid=(_NP // _BR,),
        in_specs=[
            pl.BlockSpec((_NC, _BR, _HD), lambda i: (0, i, 0)),
            pl.BlockSpec((_BR, _CW), lambda i: (i, 0)),
            pl.BlockSpec((_BR, _D), lambda i: (i, 0)),
        ],
        out_specs=pl.BlockSpec((_BR, _D), lambda i: (i, 0)),
        out_shape=jax.ShapeDtypeStruct((_N, _D), _f32),
    )(s, cnt, q)


def kernel(x, edge_index, Wl1, Wr1, b1, Wl2, Wr2, b2, Wl3, Wr3, b3):
    src = edge_index[0].astype(jnp.int32).reshape(_NS, _NCHUNK, _CHUNK)
    dst = edge_index[1].astype(jnp.int32).reshape(_NS, _NCHUNK, _CHUNK)

    zo = jnp.concatenate([jnp.zeros((_ZR, _CW), _f32),
                          jnp.ones((_CHUNK, _CW), _f32)])
    p1, q1 = _tc_in(x, Wl1, Wr1, b1)
    s1, cnt = _make_sc_segsum(True)(p1, src, dst, zo)
    p2, q2 = _tc_mid(s1, cnt, q1, Wl2, Wr2, b2)
    (s2,) = _make_sc_segsum(False)(p2, src, dst)
    p3, q3 = _tc_mid(s2, cnt, q2, Wl3, Wr3, b3)
    (s3,) = _make_sc_segsum(False)(p3, src, dst)
    return _tc_out(s3, cnt, q3)


# speedup vs baseline: 11.8824x; 1.0795x over previous
import functools

import jax
import jax.numpy as jnp
from jax import lax
from jax.experimental import pallas as pl
from jax.experimental.pallas import tpu as pltpu
from jax.experimental.pallas import tpu_sc as plsc

_N = 10000
_NP = 10240
_E = 320000
_D = 128
_NC = 2
_HD = _D // _NC
_NS = 16
_EPT = _E // _NS
_CHUNK = 125
_NCHUNK = _EPT // _CHUNK
_RPT = _NP // _NS
_ZR = 128
_CW = 8
_NPC = 10112
_RPTC = _NPC // _NS
_LANES = 16

_f32 = jnp.float32


def _nbuf(with_cnt):
    return 4


def _fill_rows(ref, nrows, ncol16, value):
    v = jnp.full((_LANES,), value, _f32)

    def body(k, _):
        i = k // ncol16
        j = k % ncol16
        ref[i, pl.ds(j * _LANES, _LANES)] = v
        return 0

    lax.fori_loop(0, nrows * ncol16, body, 0)


def _segsum_body(with_cnt, *refs):
    nbuf = _nbuf(with_cnt)
    if with_cnt:
        (p_hbm, src_hbm, dst_hbm, zo_hbm, out_hbm, cnt_hbm,
         src_v, dst_v, *rest) = refs
        rows = rest[:nbuf]
        zbuf_v, zo_v, acc_s, cnt_s = rest[nbuf:nbuf + 4]
        sems = rest[nbuf + 4:]
    else:
        (p_hbm, src_hbm, dst_hbm, out_hbm,
         src_v, dst_v, *rest) = refs
        rows = rest[:nbuf]
        zbuf_v, acc_s = rest[nbuf:nbuf + 2]
        sems = rest[nbuf + 2:]
    bufs = tuple(zip(rows, sems))

    c = lax.axis_index("c")
    s = lax.axis_index("s")

    rpt = _RPTC if with_cnt else _RPT
    pieces = [(k * _ZR, min(_ZR, rpt - k * _ZR))
              for k in range((rpt + _ZR - 1) // _ZR)]
    _fill_rows(zbuf_v, _ZR, _HD // _LANES, 0.0)
    for off, ln in pieces:
        pltpu.sync_copy(zbuf_v.at[pl.ds(0, ln)],
                        acc_s.at[pl.ds(s * rpt + off, ln)])
    if with_cnt:
        pltpu.sync_copy(zo_hbm, zo_v)
        ones_v = zo_v.at[pl.ds(_ZR, _CHUNK)]

        @pl.when(c == 0)
        def _():
            for off, ln in pieces:
                pltpu.sync_copy(zo_v.at[pl.ds(0, ln)],
                                cnt_s.at[pl.ds(s * rpt + off, ln)])

    plsc.subcore_barrier()

    pltpu.sync_copy(src_hbm.at[s], src_v)
    pltpu.sync_copy(dst_hbm.at[s], dst_v)
    my_p = p_hbm.at[c]

    def gath(j, b):
        pltpu.async_copy(my_p.at[src_v.at[j]], bufs[b][0], bufs[b][1])

    def gwait(b):
        pltpu.make_async_copy(my_p.at[src_v.at[0]], bufs[b][0], bufs[b][1]).wait()

    def scat(j, b):
        pltpu.sync_copy(bufs[b][0], acc_s.at[dst_v.at[j]], add=True)
        if with_cnt:
            @pl.when(c == 0)
            def _():
                pltpu.sync_copy(ones_v, cnt_s.at[dst_v.at[j]], add=True)

    for b in range(nbuf - 1):
        gath(b, b)

    def body(k, _):
        j = k * nbuf
        for u in range(nbuf):
            gath(j + nbuf - 1 + u, (nbuf - 1 + u) % nbuf)
            gwait(u)
            scat(j + u, u)
        return 0

    lax.fori_loop(0, (_NCHUNK - nbuf) // nbuf, body, 0)
    j = _NCHUNK - nbuf
    gath(j + nbuf - 1, nbuf - 1)
    for u in range(nbuf):
        gwait(u)
        scat(j + u, u)

    plsc.subcore_barrier()

    pltpu.sync_copy(acc_s.at[pl.ds(s * rpt, rpt)],
                    out_hbm.at[c].at[pl.ds(s * rpt, rpt)])
    if with_cnt:
        @pl.when((c == 0) & (s == 0))
        def _():
            pltpu.sync_copy(cnt_s, cnt_hbm)


@functools.lru_cache(maxsize=None)
def _make_sc_segsum(with_cnt):
    mesh = plsc.VectorSubcoreMesh(core_axis_name="c", subcore_axis_name="s",
                                  num_cores=_NC, num_subcores=_NS)
    nbuf = _nbuf(with_cnt)
    np_acc = _NPC if with_cnt else _NP
    out_type = [jax.ShapeDtypeStruct((_NC, np_acc, _HD), _f32)]
    scratch = [
        pltpu.VMEM((_NCHUNK, _CHUNK), jnp.int32),
        pltpu.VMEM((_NCHUNK, _CHUNK), jnp.int32),
    ]
    for _ in range(nbuf):
        scratch.append(pltpu.VMEM((_CHUNK, _HD), _f32))
    scratch.append(pltpu.VMEM((_ZR, _HD), _f32))
    if with_cnt:
        out_type.append(jax.ShapeDtypeStruct((_NPC, _CW), _f32))
        scratch.append(pltpu.VMEM((_ZR + _CHUNK, _CW), _f32))
    scratch.append(pltpu.VMEM_SHARED((np_acc, _HD), _f32))
    if with_cnt:
        scratch.append(pltpu.VMEM_SHARED((_NPC, _CW), _f32))
    for _ in range(nbuf):
        scratch.append(pltpu.SemaphoreType.DMA)

    return pl.kernel(
        functools.partial(_segsum_body, with_cnt),
        out_type=tuple(out_type),
        mesh=mesh,
        scratch_types=tuple(scratch),
        compiler_params=pltpu.CompilerParams(use_tc_tiling_on_sc=False),
        name="sc_segsum_cnt" if with_cnt else "sc_segsum",
    )


_BR = 512


def _split(p):
    return jnp.stack([p[:, :_HD], p[:, _HD:]])


def _tc_in_body(x_ref, wl_ref, wr_ref, b_ref, p_ref, q_ref):
    x = x_ref[...]
    p_ref[...] = _split(jnp.dot(x, wl_ref[...], preferred_element_type=_f32))
    q_ref[...] = jnp.dot(x, wr_ref[...], preferred_element_type=_f32) + b_ref[...]


def _tc_in(x, wl, wr, b):
    return pl.pallas_call(
        _tc_in_body,
        grid=(_NP // _BR,),
        in_specs=[
            pl.BlockSpec((_BR, _D), lambda i: (i, 0)),
            pl.BlockSpec((_D, _D), lambda i: (0, 0)),
            pl.BlockSpec((_D, _D), lambda i: (0, 0)),
            pl.BlockSpec((1, _D), lambda i: (0, 0)),
        ],
        out_specs=[
            pl.BlockSpec((_NC, _BR, _HD), lambda i: (0, i, 0)),
            pl.BlockSpec((_BR, _D), lambda i: (i, 0)),
        ],
        out_shape=[
            jax.ShapeDtypeStruct((_NC, _NP, _HD), _f32),
            jax.ShapeDtypeStruct((_NP, _D), _f32),
        ],
    )(x, wl, wr, b.reshape(1, _D))


def _relu_mean(s_ref, cnt_ref, q_ref):
    ssum = jnp.concatenate([s_ref[0], s_ref[1]], axis=1)
    cnt = cnt_ref[:, 0:1]
    mean = ssum / jnp.maximum(cnt, 1.0)
    return jnp.maximum(mean + q_ref[...], 0.0)


def _tc_mid_body(s_ref, cnt_ref, q_ref, wl_ref, wr_ref, b_ref, p_ref, q2_ref):
    h = _relu_mean(s_ref, cnt_ref, q_ref)
    p_ref[...] = _split(jnp.dot(h, wl_ref[...], preferred_element_type=_f32))
    q2_ref[...] = jnp.dot(h, wr_ref[...], preferred_element_type=_f32) + b_ref[...]


def _tc_mid(s, cnt, q, wl, wr, b):
    return pl.pallas_call(
        _tc_mid_body,
        grid=(_NP // _BR,),
        in_specs=[
            pl.BlockSpec((_NC, _BR, _HD), lambda i: (0, i, 0)),
            pl.BlockSpec((_BR, _CW), lambda i: (i, 0)),
            pl.BlockSpec((_BR, _D), lambda i: (i, 0)),
            pl.BlockSpec((_D, _D), lambda i: (0, 0)),
            pl.BlockSpec((_D, _D), lambda i: (0, 0)),
            pl.BlockSpec((1, _D), lambda i: (0, 0)),
        ],
        out_specs=[
            pl.BlockSpec((_NC, _BR, _HD), lambda i: (0, i, 0)),
            pl.BlockSpec((_BR, _D), lambda i: (i, 0)),
        ],
        out_shape=[
            jax.ShapeDtypeStruct((_NC, _NP, _HD), _f32),
            jax.ShapeDtypeStruct((_NP, _D), _f32),
        ],
    )(s, cnt, q, wl, wr, b.reshape(1, _D))


def _tc_out_body(s_ref, cnt_ref, q_ref, h_ref):
    h_ref[...] = _relu_mean(s_ref, cnt_ref, q_ref)


def _tc_out(s, cnt, q):
    return pl.pallas_call(
        _tc_out_body,
        grid=(_NP // _BR,),
        in_specs=[
            pl.BlockSpec((_NC, _BR, _HD), lambda i: (0, i, 0)),
            pl.BlockSpec((_BR, _CW), lambda i: (i, 0)),
            pl.BlockSpec((_BR, _D), lambda i: (i, 0)),
        ],
        out_specs=pl.BlockSpec((_BR, _D), lambda i: (i, 0)),
        out_shape=jax.ShapeDtypeStruct((_N, _D), _f32),
    )(s, cnt, q)


def kernel(x, edge_index, Wl1, Wr1, b1, Wl2, Wr2, b2, Wl3, Wr3, b3):
    src = edge_index[0].astype(jnp.int32).reshape(_NS, _NCHUNK, _CHUNK)
    dst = edge_index[1].astype(jnp.int32).reshape(_NS, _NCHUNK, _CHUNK)

    zo = jnp.concatenate([jnp.zeros((_ZR, _CW), _f32),
                          jnp.ones((_CHUNK, _CW), _f32)])
    p1, q1 = _tc_in(x, Wl1, Wr1, b1)
    s1, cnt = _make_sc_segsum(True)(p1, src, dst, zo)
    p2, q2 = _tc_mid(s1, cnt, q1, Wl2, Wr2, b2)
    (s2,) = _make_sc_segsum(False)(p2, src, dst)
    p3, q3 = _tc_mid(s2, cnt, q2, Wl3, Wr3, b3)
    (s3,) = _make_sc_segsum(False)(p3, src, dst)
    return _tc_out(s3, cnt, q3)


# speedup vs baseline: 13.7083x; 1.1537x over previous
import functools

import jax
import jax.numpy as jnp
from jax import lax
from jax.experimental import pallas as pl
from jax.experimental.pallas import tpu as pltpu
from jax.experimental.pallas import tpu_sc as plsc

_N = 10000
_NP = 10240
_E = 320000
_D = 128
_NC = 2
_HD = _D // _NC
_NS = 16
_EPT = _E // _NS
_CHUNK = 125
_NCHUNK = _EPT // _CHUNK
_RPT = _NP // _NS
_ZR = 128
_CW = 8
_NPC = 10112
_RPTC = _NPC // _NS
_LANES = 16

_f32 = jnp.float32


def _nbuf(with_cnt):
    return 4


def _fill_rows(ref, nrows, ncol16, value):
    v = jnp.full((_LANES,), value, _f32)

    def body(k, _):
        i = k // ncol16
        j = k % ncol16
        ref[i, pl.ds(j * _LANES, _LANES)] = v
        return 0

    lax.fori_loop(0, nrows * ncol16, body, 0)


def _segsum_body(with_cnt, *refs):
    nbuf = _nbuf(with_cnt)
    if with_cnt:
        (p_hbm, src_hbm, dst_hbm, zo_hbm, out_hbm, cnt_hbm,
         src_v, dst_v, *rest) = refs
        rows = rest[:nbuf]
        zbuf_v, zo_v, acc_s, cnt_s = rest[nbuf:nbuf + 4]
        sems = rest[nbuf + 4:]
    else:
        (p_hbm, src_hbm, dst_hbm, out_hbm,
         src_v, dst_v, *rest) = refs
        rows = rest[:nbuf]
        zbuf_v, acc_s = rest[nbuf:nbuf + 2]
        sems = rest[nbuf + 2:]
    bufs = tuple(zip(rows, sems))

    c = lax.axis_index("c")
    s = lax.axis_index("s")

    rpt = _RPTC if with_cnt else _RPT
    pieces = [(k * _ZR, min(_ZR, rpt - k * _ZR))
              for k in range((rpt + _ZR - 1) // _ZR)]
    _fill_rows(zbuf_v, _ZR, _HD // _LANES, 0.0)
    for off, ln in pieces:
        pltpu.sync_copy(zbuf_v.at[pl.ds(0, ln)],
                        acc_s.at[pl.ds(s * rpt + off, ln)])
    if with_cnt:
        pltpu.sync_copy(zo_hbm, zo_v)
        ones_v = zo_v.at[pl.ds(_ZR, _CHUNK)]

        @pl.when(c == 0)
        def _():
            for off, ln in pieces:
                pltpu.sync_copy(zo_v.at[pl.ds(0, ln)],
                                cnt_s.at[pl.ds(s * rpt + off, ln)])

    plsc.subcore_barrier()

    pltpu.sync_copy(src_hbm.at[c].at[s], src_v)
    pltpu.sync_copy(dst_hbm.at[s], dst_v)
    my_p = p_hbm

    def gath(j, b):
        pltpu.async_copy(my_p.at[src_v.at[j]], bufs[b][0], bufs[b][1])

    def gwait(b):
        pltpu.make_async_copy(my_p.at[src_v.at[0]], bufs[b][0], bufs[b][1]).wait()

    def scat(j, b):
        pltpu.sync_copy(bufs[b][0], acc_s.at[dst_v.at[j]], add=True)
        if with_cnt:
            @pl.when(c == 0)
            def _():
                pltpu.sync_copy(ones_v, cnt_s.at[dst_v.at[j]], add=True)

    for b in range(nbuf - 1):
        gath(b, b)

    def body(k, _):
        j = k * nbuf
        for u in range(nbuf):
            gath(j + nbuf - 1 + u, (nbuf - 1 + u) % nbuf)
            gwait(u)
            scat(j + u, u)
        return 0

    lax.fori_loop(0, (_NCHUNK - nbuf) // nbuf, body, 0)
    j = _NCHUNK - nbuf
    gath(j + nbuf - 1, nbuf - 1)
    for u in range(nbuf):
        gwait(u)
        scat(j + u, u)

    plsc.subcore_barrier()

    @pl.when(c == 0)
    def _():
        pltpu.sync_copy(acc_s.at[pl.ds(s * rpt, rpt)],
                        out_hbm.at[pl.ds(s * rpt, rpt), pl.ds(0, _HD)])

    @pl.when(c == 1)
    def _():
        pltpu.sync_copy(acc_s.at[pl.ds(s * rpt, rpt)],
                        out_hbm.at[pl.ds(s * rpt, rpt), pl.ds(_HD, _HD)])
    if with_cnt:
        @pl.when((c == 0) & (s == 0))
        def _():
            pltpu.sync_copy(cnt_s, cnt_hbm)


@functools.lru_cache(maxsize=None)
def _make_sc_segsum(with_cnt):
    mesh = plsc.VectorSubcoreMesh(core_axis_name="c", subcore_axis_name="s",
                                  num_cores=_NC, num_subcores=_NS)
    nbuf = _nbuf(with_cnt)
    np_acc = _NPC if with_cnt else _NP
    out_type = [jax.ShapeDtypeStruct((np_acc, _D), _f32)]
    scratch = [
        pltpu.VMEM((_NCHUNK, _CHUNK), jnp.int32),
        pltpu.VMEM((_NCHUNK, _CHUNK), jnp.int32),
    ]
    for _ in range(nbuf):
        scratch.append(pltpu.VMEM((_CHUNK, _HD), _f32))
    scratch.append(pltpu.VMEM((_ZR, _HD), _f32))
    if with_cnt:
        out_type.append(jax.ShapeDtypeStruct((_NPC, _CW), _f32))
        scratch.append(pltpu.VMEM((_ZR + _CHUNK, _CW), _f32))
    scratch.append(pltpu.VMEM_SHARED((np_acc, _HD), _f32))
    if with_cnt:
        scratch.append(pltpu.VMEM_SHARED((_NPC, _CW), _f32))
    for _ in range(nbuf):
        scratch.append(pltpu.SemaphoreType.DMA)

    return pl.kernel(
        functools.partial(_segsum_body, with_cnt),
        out_type=tuple(out_type),
        mesh=mesh,
        scratch_types=tuple(scratch),
        compiler_params=pltpu.CompilerParams(use_tc_tiling_on_sc=False),
        name="sc_segsum_cnt" if with_cnt else "sc_segsum",
    )


_BR = 512


def _tc_in_body(x_ref, wl_ref, wr_ref, b_ref, p_ref, q_ref):
    x = x_ref[...]
    p_ref[...] = jnp.dot(x, wl_ref[...], preferred_element_type=_f32)
    q_ref[...] = jnp.dot(x, wr_ref[...], preferred_element_type=_f32) + b_ref[...]


def _tc_in(x, wl, wr, b):
    return pl.pallas_call(
        _tc_in_body,
        grid=(_NP // _BR,),
        in_specs=[
            pl.BlockSpec((_BR, _D), lambda i: (i, 0)),
            pl.BlockSpec((_D, _D), lambda i: (0, 0)),
            pl.BlockSpec((_D, _D), lambda i: (0, 0)),
            pl.BlockSpec((1, _D), lambda i: (0, 0)),
        ],
        out_specs=[
            pl.BlockSpec((_BR, _D), lambda i: (i, 0)),
            pl.BlockSpec((_BR, _D), lambda i: (i, 0)),
        ],
        out_shape=[
            jax.ShapeDtypeStruct((_NP, _D), _f32),
            jax.ShapeDtypeStruct((_NP, _D), _f32),
        ],
    )(x, wl, wr, b.reshape(1, _D))


def _relu_mean(s_ref, cnt_ref, q_ref):
    ssum = s_ref[...]
    cnt = cnt_ref[:, 0:1]
    mean = ssum / jnp.maximum(cnt, 1.0)
    return jnp.maximum(mean + q_ref[...], 0.0)


def _tc_mid_body(s_ref, cnt_ref, q_ref, wl_ref, wr_ref, b_ref, p_ref, q2_ref):
    h = _relu_mean(s_ref, cnt_ref, q_ref)
    p_ref[...] = jnp.dot(h, wl_ref[...], preferred_element_type=_f32)
    q2_ref[...] = jnp.dot(h, wr_ref[...], preferred_element_type=_f32) + b_ref[...]


def _tc_mid(s, cnt, q, wl, wr, b):
    return pl.pallas_call(
        _tc_mid_body,
        grid=(_NP // _BR,),
        in_specs=[
            pl.BlockSpec((_BR, _D), lambda i: (i, 0)),
            pl.BlockSpec((_BR, _CW), lambda i: (i, 0)),
            pl.BlockSpec((_BR, _D), lambda i: (i, 0)),
            pl.BlockSpec((_D, _D), lambda i: (0, 0)),
            pl.BlockSpec((_D, _D), lambda i: (0, 0)),
            pl.BlockSpec((1, _D), lambda i: (0, 0)),
        ],
        out_specs=[
            pl.BlockSpec((_BR, _D), lambda i: (i, 0)),
            pl.BlockSpec((_BR, _D), lambda i: (i, 0)),
        ],
        out_shape=[
            jax.ShapeDtypeStruct((_NP, _D), _f32),
            jax.ShapeDtypeStruct((_NP, _D), _f32),
        ],
    )(s, cnt, q, wl, wr, b.reshape(1, _D))


def _tc_out_body(s_ref, cnt_ref, q_ref, h_ref):
    h_ref[...] = _relu_mean(s_ref, cnt_ref, q_ref)


def _tc_out(s, cnt, q):
    return pl.pallas_call(
        _tc_out_body,
        grid=(_NP // _BR,),
        in_specs=[
            pl.BlockSpec((_BR, _D), lambda i: (i, 0)),
            pl.BlockSpec((_BR, _CW), lambda i: (i, 0)),
            pl.BlockSpec((_BR, _D), lambda i: (i, 0)),
        ],
        out_specs=pl.BlockSpec((_BR, _D), lambda i: (i, 0)),
        out_shape=jax.ShapeDtypeStruct((_N, _D), _f32),
    )(s, cnt, q)


def kernel(x, edge_index, Wl1, Wr1, b1, Wl2, Wr2, b2, Wl3, Wr3, b3):
    sr = edge_index[0].astype(jnp.int32).reshape(_NS, _NCHUNK, _CHUNK)
    src2 = jnp.stack([2 * sr, 2 * sr + 1])
    dst = edge_index[1].astype(jnp.int32).reshape(_NS, _NCHUNK, _CHUNK)

    zo = jnp.concatenate([jnp.zeros((_ZR, _CW), _f32),
                          jnp.ones((_CHUNK, _CW), _f32)])
    p1, q1 = _tc_in(x, Wl1, Wr1, b1)
    s1, cnt = _make_sc_segsum(True)(p1.reshape(2 * _NP, _HD), src2, dst, zo)
    p2, q2 = _tc_mid(s1, cnt, q1, Wl2, Wr2, b2)
    (s2,) = _make_sc_segsum(False)(p2.reshape(2 * _NP, _HD), src2, dst)
    p3, q3 = _tc_mid(s2, cnt, q2, Wl3, Wr3, b3)
    (s3,) = _make_sc_segsum(False)(p3.reshape(2 * _NP, _HD), src2, dst)
    return _tc_out(s3, cnt, q3)


# speedup vs baseline: 14.5027x; 1.0579x over previous
import functools

import jax
import jax.numpy as jnp
from jax import lax
from jax.experimental import pallas as pl
from jax.experimental.pallas import tpu as pltpu
from jax.experimental.pallas import tpu_sc as plsc

_N = 10000
_NP = 10240
_E = 320000
_D = 128
_NC = 2
_HD = _D // _NC
_NS = 16
_EPT = _E // _NS
_CHUNK = 125
_NCHUNK = _EPT // _CHUNK
_RPT = _NP // _NS
_ZR = 128
_CW = 8
_NPC = 10112
_RPTC = _NPC // _NS
_LANES = 16

_f32 = jnp.float32


def _nbuf(with_cnt):
    return 4


def _fill_rows(ref, nrows, ncol16, value):
    v = jnp.full((_LANES,), value, _f32)

    def body(k, _):
        i = k // ncol16
        j = k % ncol16
        ref[i, pl.ds(j * _LANES, _LANES)] = v
        return 0

    lax.fori_loop(0, nrows * ncol16, body, 0)


def _segsum_body(with_cnt, *refs):
    nbuf = _nbuf(with_cnt)
    if with_cnt:
        (p_hbm, src_hbm, dst_hbm, zo_hbm, out_hbm, cnt_hbm,
         src_v, dst_v, *rest) = refs
        rows = rest[:nbuf]
        zbuf_v, zo_v, acc_s, cnt_s = rest[nbuf:nbuf + 4]
        sems = rest[nbuf + 4:]
    else:
        (p_hbm, src_hbm, dst_hbm, out_hbm,
         src_v, dst_v, *rest) = refs
        rows = rest[:nbuf]
        zbuf_v, acc_s = rest[nbuf:nbuf + 2]
        sems = rest[nbuf + 2:]
    bufs = tuple(zip(rows, sems))

    c = lax.axis_index("c")
    s = lax.axis_index("s")

    rpt = _RPTC if with_cnt else _RPT
    pieces = [(k * _ZR, min(_ZR, rpt - k * _ZR))
              for k in range((rpt + _ZR - 1) // _ZR)]
    _fill_rows(zbuf_v, _ZR, _HD // _LANES, 0.0)
    for off, ln in pieces:
        pltpu.sync_copy(zbuf_v.at[pl.ds(0, ln)],
                        acc_s.at[pl.ds(s * rpt + off, ln)])
    if with_cnt:
        pltpu.sync_copy(zo_hbm, zo_v)
        ones_v = zo_v.at[pl.ds(_ZR, _CHUNK)]

        @pl.when(c == 0)
        def _():
            for off, ln in pieces:
                pltpu.sync_copy(zo_v.at[pl.ds(0, ln)],
                                cnt_s.at[pl.ds(s * rpt + off, ln)])

    plsc.subcore_barrier()

    pltpu.sync_copy(src_hbm.at[c].at[s], src_v)
    pltpu.sync_copy(dst_hbm.at[s], dst_v)
    my_p = p_hbm

    def gath(j, b):
        pltpu.async_copy(my_p.at[src_v.at[j]], bufs[b][0], bufs[b][1])

    def gwait(b):
        pltpu.make_async_copy(my_p.at[src_v.at[0]], bufs[b][0], bufs[b][1]).wait()

    def scat(j, b):
        pltpu.sync_copy(bufs[b][0], acc_s.at[dst_v.at[j]], add=True)
        if with_cnt:
            @pl.when(c == 0)
            def _():
                pltpu.sync_copy(ones_v, cnt_s.at[dst_v.at[j]], add=True)

    for b in range(nbuf - 1):
        gath(b, b)

    def body(k, _):
        j = k * nbuf
        for u in range(nbuf):
            gath(j + nbuf - 1 + u, (nbuf - 1 + u) % nbuf)
            gwait(u)
            scat(j + u, u)
        return 0

    lax.fori_loop(0, (_NCHUNK - nbuf) // nbuf, body, 0)
    j = _NCHUNK - nbuf
    gath(j + nbuf - 1, nbuf - 1)
    for u in range(nbuf):
        gwait(u)
        scat(j + u, u)

    plsc.subcore_barrier()

    @pl.when(c == 0)
    def _():
        pltpu.sync_copy(acc_s.at[pl.ds(s * rpt, rpt)],
                        out_hbm.at[pl.ds(s * rpt, rpt), pl.ds(0, _HD)])

    @pl.when(c == 1)
    def _():
        pltpu.sync_copy(acc_s.at[pl.ds(s * rpt, rpt)],
                        out_hbm.at[pl.ds(s * rpt, rpt), pl.ds(_HD, _HD)])
    if with_cnt:
        @pl.when((c == 0) & (s == 0))
        def _():
            pltpu.sync_copy(cnt_s, cnt_hbm)


@functools.lru_cache(maxsize=None)
def _make_sc_segsum(with_cnt):
    mesh = plsc.VectorSubcoreMesh(core_axis_name="c", subcore_axis_name="s",
                                  num_cores=_NC, num_subcores=_NS)
    nbuf = _nbuf(with_cnt)
    np_acc = _NPC if with_cnt else _NP
    out_type = [jax.ShapeDtypeStruct((np_acc, _D), _f32)]
    scratch = [
        pltpu.VMEM((_NCHUNK, _CHUNK), jnp.int32),
        pltpu.VMEM((_NCHUNK, _CHUNK), jnp.int32),
    ]
    for _ in range(nbuf):
        scratch.append(pltpu.VMEM((_CHUNK, _HD), _f32))
    scratch.append(pltpu.VMEM((_ZR, _HD), _f32))
    if with_cnt:
        out_type.append(jax.ShapeDtypeStruct((_NPC, _CW), _f32))
        scratch.append(pltpu.VMEM((_ZR + _CHUNK, _CW), _f32))
    scratch.append(pltpu.VMEM_SHARED((np_acc, _HD), _f32))
    if with_cnt:
        scratch.append(pltpu.VMEM_SHARED((_NPC, _CW), _f32))
    for _ in range(nbuf):
        scratch.append(pltpu.SemaphoreType.DMA)

    return pl.kernel(
        functools.partial(_segsum_body, with_cnt),
        out_type=tuple(out_type),
        mesh=mesh,
        scratch_types=tuple(scratch),
        compiler_params=pltpu.CompilerParams(use_tc_tiling_on_sc=False),
        name="sc_segsum_cnt" if with_cnt else "sc_segsum",
    )


_BR = 1024


def _tc_in_body(x_ref, w_ref, b_ref, p_ref, q_ref):
    y = jnp.dot(x_ref[...], w_ref[...], preferred_element_type=_f32)
    p_ref[...] = y[:, :_D]
    q_ref[...] = y[:, _D:] + b_ref[...]


def _tc_in(x, wl, wr, b):
    return pl.pallas_call(
        _tc_in_body,
        grid=(_NP // _BR,),
        in_specs=[
            pl.BlockSpec((_BR, _D), lambda i: (i, 0)),
            pl.BlockSpec((_D, 2 * _D), lambda i: (0, 0)),
            pl.BlockSpec((1, _D), lambda i: (0, 0)),
        ],
        out_specs=[
            pl.BlockSpec((_BR, _D), lambda i: (i, 0)),
            pl.BlockSpec((_BR, _D), lambda i: (i, 0)),
        ],
        out_shape=[
            jax.ShapeDtypeStruct((_NP, _D), _f32),
            jax.ShapeDtypeStruct((_NP, _D), _f32),
        ],
    )(x, jnp.concatenate([wl, wr], axis=1), b.reshape(1, _D))


def _relu_mean(s_ref, cnt_ref, q_ref):
    ssum = s_ref[...]
    cnt = cnt_ref[:, 0:1]
    mean = ssum / jnp.maximum(cnt, 1.0)
    return jnp.maximum(mean + q_ref[...], 0.0)


def _tc_mid_body(s_ref, cnt_ref, q_ref, w_ref, b_ref, p_ref, q2_ref):
    h = _relu_mean(s_ref, cnt_ref, q_ref)
    y = jnp.dot(h, w_ref[...], preferred_element_type=_f32)
    p_ref[...] = y[:, :_D]
    q2_ref[...] = y[:, _D:] + b_ref[...]


def _tc_mid(s, cnt, q, wl, wr, b):
    return pl.pallas_call(
        _tc_mid_body,
        grid=(_NP // _BR,),
        in_specs=[
            pl.BlockSpec((_BR, _D), lambda i: (i, 0)),
            pl.BlockSpec((_BR, _CW), lambda i: (i, 0)),
            pl.BlockSpec((_BR, _D), lambda i: (i, 0)),
            pl.BlockSpec((_D, 2 * _D), lambda i: (0, 0)),
            pl.BlockSpec((1, _D), lambda i: (0, 0)),
        ],
        out_specs=[
            pl.BlockSpec((_BR, _D), lambda i: (i, 0)),
            pl.BlockSpec((_BR, _D), lambda i: (i, 0)),
        ],
        out_shape=[
            jax.ShapeDtypeStruct((_NP, _D), _f32),
            jax.ShapeDtypeStruct((_NP, _D), _f32),
        ],
    )(s, cnt, q, jnp.concatenate([wl, wr], axis=1), b.reshape(1, _D))


def _tc_out_body(s_ref, cnt_ref, q_ref, h_ref):
    h_ref[...] = _relu_mean(s_ref, cnt_ref, q_ref)


def _tc_out(s, cnt, q):
    return pl.pallas_call(
        _tc_out_body,
        grid=(_NP // _BR,),
        in_specs=[
            pl.BlockSpec((_BR, _D), lambda i: (i, 0)),
            pl.BlockSpec((_BR, _CW), lambda i: (i, 0)),
            pl.BlockSpec((_BR, _D), lambda i: (i, 0)),
        ],
        out_specs=pl.BlockSpec((_BR, _D), lambda i: (i, 0)),
        out_shape=jax.ShapeDtypeStruct((_N, _D), _f32),
    )(s, cnt, q)


def kernel(x, edge_index, Wl1, Wr1, b1, Wl2, Wr2, b2, Wl3, Wr3, b3):
    sr = edge_index[0].astype(jnp.int32).reshape(_NS, _NCHUNK, _CHUNK)
    src2 = jnp.stack([2 * sr, 2 * sr + 1])
    dst = edge_index[1].astype(jnp.int32).reshape(_NS, _NCHUNK, _CHUNK)

    zo = jnp.concatenate([jnp.zeros((_ZR, _CW), _f32),
                          jnp.ones((_CHUNK, _CW), _f32)])
    p1, q1 = _tc_in(x, Wl1, Wr1, b1)
    s1, cnt = _make_sc_segsum(True)(p1.reshape(2 * _NP, _HD), src2, dst, zo)
    p2, q2 = _tc_mid(s1, cnt, q1, Wl2, Wr2, b2)
    (s2,) = _make_sc_segsum(False)(p2.reshape(2 * _NP, _HD), src2, dst)
    p3, q3 = _tc_mid(s2, cnt, q2, Wl3, Wr3, b3)
    (s3,) = _make_sc_segsum(False)(p3.reshape(2 * _NP, _HD), src2, dst)
    return _tc_out(s3, cnt, q3)
